# baseline (speedup 1.0000x reference)
import functools

import jax
import jax.numpy as jnp
from jax.experimental import pallas as pl
from jax.experimental.pallas import tpu as pltpu

_BM = 400
_S = 3


def _gcn_stream_kernel(nchunk, x_ref, adj_hbm, w_ref, b_ref, o_ref, buf, sems):
    x = x_ref[...]
    w = w_ref[...]
    b = b_ref[...]

    def start_copy(chunk, slot):
        pltpu.make_async_copy(
            adj_hbm.at[pl.ds(chunk * _BM, _BM), :],
            buf.at[slot],
            sems.at[slot],
        ).start()

    for s in range(min(_S, nchunk)):
        start_copy(s, s)

    def step(i, carry):
        slot = jax.lax.rem(i, _S)
        pltpu.make_async_copy(
            adj_hbm.at[pl.ds(i * _BM, _BM), :],
            buf.at[slot],
            sems.at[slot],
        ).wait()
        t = jnp.dot(buf[slot], x, preferred_element_type=jnp.float32)
        o_ref[pl.ds(i * _BM, _BM), :] = (
            jnp.dot(t, w, preferred_element_type=jnp.float32) + b
        )

        nxt = i + _S

        @pl.when(nxt < nchunk)
        def _():
            start_copy(nxt, slot)

        return carry

    jax.lax.fori_loop(0, nchunk, step, 0)


@jax.jit
def kernel(input, adj, W, b):
    n, d_in = input.shape
    d_out = W.shape[1]
    m = adj.shape[0]
    assert m % _BM == 0
    b2 = b.reshape(1, d_out)
    return pl.pallas_call(
        functools.partial(_gcn_stream_kernel, m // _BM),
        in_specs=[
            pl.BlockSpec(memory_space=pltpu.VMEM),
            pl.BlockSpec(memory_space=pl.ANY),
            pl.BlockSpec(memory_space=pltpu.VMEM),
            pl.BlockSpec(memory_space=pltpu.VMEM),
        ],
        out_specs=pl.BlockSpec(memory_space=pltpu.VMEM),
        out_shape=jax.ShapeDtypeStruct((m, d_out), jnp.float32),
        scratch_shapes=[
            pltpu.VMEM((_S, _BM, n), jnp.float32),
            pltpu.SemaphoreType.DMA((_S,)),
        ],
        compiler_params=pltpu.CompilerParams(
            vmem_limit_bytes=64 * 1024 * 1024,
        ),
    )(input, adj, W, b2)


# speedup vs baseline: 1.0011x; 1.0011x over previous
import functools

import jax
import jax.numpy as jnp
from jax.experimental import pallas as pl
from jax.experimental.pallas import tpu as pltpu

_BM = 200
_S = 5


def _gcn_stream_kernel(nchunk, x_ref, adj_hbm, w_ref, b_ref, o_ref, buf, sems):
    x = x_ref[...]
    w = w_ref[...]
    b = b_ref[...]

    def start_copy(chunk, slot):
        pltpu.make_async_copy(
            adj_hbm.at[pl.ds(chunk * _BM, _BM), :],
            buf.at[slot],
            sems.at[slot],
        ).start()

    for s in range(min(_S, nchunk)):
        start_copy(s, s)

    def step(g, carry):
        for s in range(_S):
            i = g * _S + s
            pltpu.make_async_copy(
                adj_hbm.at[pl.ds(i * _BM, _BM), :],
                buf.at[s],
                sems.at[s],
            ).wait()
            t = jnp.dot(buf[s], x, preferred_element_type=jnp.float32)
            o_ref[pl.ds(i * _BM, _BM), :] = (
                jnp.dot(t, w, preferred_element_type=jnp.float32) + b
            )

            nxt = i + _S

            @pl.when(nxt < nchunk)
            def _():
                start_copy(nxt, s)

        return carry

    assert nchunk % _S == 0
    jax.lax.fori_loop(0, nchunk // _S, step, 0)


@jax.jit
def kernel(input, adj, W, b):
    n, d_in = input.shape
    d_out = W.shape[1]
    m = adj.shape[0]
    assert m % _BM == 0
    b2 = b.reshape(1, d_out)
    return pl.pallas_call(
        functools.partial(_gcn_stream_kernel, m // _BM),
        in_specs=[
            pl.BlockSpec(memory_space=pltpu.VMEM),
            pl.BlockSpec(memory_space=pl.ANY),
            pl.BlockSpec(memory_space=pltpu.VMEM),
            pl.BlockSpec(memory_space=pltpu.VMEM),
        ],
        out_specs=pl.BlockSpec(memory_space=pltpu.VMEM),
        out_shape=jax.ShapeDtypeStruct((m, d_out), jnp.float32),
        scratch_shapes=[
            pltpu.VMEM((_S, _BM, n), jnp.float32),
            pltpu.SemaphoreType.DMA((_S,)),
        ],
        compiler_params=pltpu.CompilerParams(
            vmem_limit_bytes=64 * 1024 * 1024,
        ),
    )(input, adj, W, b2)


# speedup vs baseline: 1.0361x; 1.0349x over previous
import functools

import jax
import jax.numpy as jnp
from jax.experimental import pallas as pl
from jax.experimental.pallas import tpu as pltpu

_BM = 400


def _gcn_block_kernel(x_ref, adj_ref, w_ref, b_ref, o_ref):
    t = jnp.dot(adj_ref[...], x_ref[...], preferred_element_type=jnp.float32)
    o_ref[...] = (
        jnp.dot(t, w_ref[...], preferred_element_type=jnp.float32) + b_ref[...]
    )


@jax.jit
def kernel(input, adj, W, b):
    n, d_in = input.shape
    d_out = W.shape[1]
    b2 = b.reshape(1, d_out)
    grid = (pl.cdiv(adj.shape[0], _BM),)
    return pl.pallas_call(
        _gcn_block_kernel,
        grid=grid,
        in_specs=[
            pl.BlockSpec((n, d_in), lambda i: (0, 0)),
            pl.BlockSpec((_BM, n), lambda i: (i, 0)),
            pl.BlockSpec((d_in, d_out), lambda i: (0, 0)),
            pl.BlockSpec((1, d_out), lambda i: (0, 0)),
        ],
        out_specs=pl.BlockSpec((_BM, d_out), lambda i: (i, 0)),
        out_shape=jax.ShapeDtypeStruct((adj.shape[0], d_out), jnp.float32),
        compiler_params=pltpu.CompilerParams(
            dimension_semantics=("parallel",),
        ),
    )(input, adj, W, b2)
